# baseline (speedup 1.0000x reference)
import functools

import jax
import jax.numpy as jnp
from jax import lax
from jax.experimental import pallas as pl
from jax.experimental.pallas import tpu as pltpu
from jax.experimental.pallas import tpu_sc as plsc

_NC = 2
_NS = 16
_LANES = 16
_CHUNK = 80
_NB = 3
_G = 8
_NGB = 3
_SUPER = _NGB * _G


def _spmm(xr, meta, wr, n_nodes):
    _, dh = xr.shape
    n = n_nodes
    ng = meta.shape[1]
    nch = ng * _G
    rpt = (n // _NS) // 8 * 8
    rem = n - _NS * rpt
    nsteady = nch - _G
    assert nsteady % _SUPER == 0 and ng >= 2

    mesh = plsc.VectorSubcoreMesh(
        core_axis_name="c", subcore_axis_name="s", num_cores=_NC, num_subcores=_NS
    )

    @functools.partial(
        pl.kernel,
        mesh=mesh,
        out_type=jax.ShapeDtypeStruct((_NC, n, dh), jnp.float32),
        scratch_types=[
            pltpu.VMEM((_NGB, _G, _CHUNK), jnp.int32),
            pltpu.VMEM((_NGB, _G, _CHUNK), jnp.int32),
            pltpu.VMEM((_NGB, _G, _CHUNK), jnp.float32),
            pltpu.VMEM((_NB, _CHUNK), jnp.int32),
            pltpu.VMEM((_NB, _CHUNK), jnp.int32),
            pltpu.VMEM((_NB, _CHUNK, dh), jnp.float32),
            pltpu.VMEM_SHARED((n, dh), jnp.float32),
            [pltpu.SemaphoreType.DMA] * _NGB,
            [pltpu.SemaphoreType.DMA] * _NB,
            [pltpu.SemaphoreType.DMA] * _NB,
        ],
    )
    def k(xr_hbm, meta_hbm, w_hbm, out_hbm, mgs, mgd, wgb, gb, db, rows, agg,
          sem_m, sem_g, sem_s):
        c = lax.axis_index("c")
        s = lax.axis_index("s")
        rbase = pl.multiple_of(s * rpt, 8)
        tbase = _NS * rpt

        def issue_meta(i, gsl):
            pltpu.async_copy(meta_hbm.at[s, i], mgs.at[gsl], sem_m[gsl])
            pltpu.async_copy(meta_hbm.at[_NS + s, i], mgd.at[gsl], sem_m[gsl])
            pltpu.async_copy(w_hbm.at[s, i], wgb.at[gsl], sem_m[gsl])

        def wait_meta(i, gsl):
            pltpu.make_async_copy(meta_hbm.at[s, i], mgs.at[gsl], sem_m[gsl]).wait()
            pltpu.make_async_copy(
                meta_hbm.at[_NS + s, i], mgd.at[gsl], sem_m[gsl]
            ).wait()
            pltpu.make_async_copy(w_hbm.at[s, i], wgb.at[gsl], sem_m[gsl]).wait()

        def prep_gather(b, gsl, kk):
            for v in range(_CHUNK // _LANES):
                sl = pl.ds(v * _LANES, _LANES)
                gb[b, sl] = mgs[gsl, kk, sl] * 2 + c
                db[b, sl] = mgd[gsl, kk, sl]
            pltpu.async_copy(xr_hbm.at[gb.at[b]], rows.at[b], sem_g[b])

        def wait_gather(b):
            pltpu.make_async_copy(xr_hbm.at[gb.at[b]], rows.at[b], sem_g[b]).wait()

        def scale(b, gsl, kk):
            def grp(g, carry):
                wg = wgb[gsl, kk, pl.ds(g * _LANES, _LANES)]
                ws = [wg[r16] for r16 in range(_LANES)]

                def feat(v, inner):
                    sl = pl.ds(v * _LANES, _LANES)
                    for r16 in range(_LANES):
                        r = g * _LANES + r16
                        rows[b, r, sl] = rows[b, r, sl] * ws[r16]
                    return inner

                lax.fori_loop(0, dh // _LANES, feat, None)
                return carry

            lax.fori_loop(0, _CHUNK // _LANES, grp, None)

        def issue_scatter(b):
            pltpu.async_copy(rows.at[b], agg.at[db.at[b]], sem_s[b], add=True)

        def wait_scatter(b):
            pltpu.make_async_copy(rows.at[b], agg.at[db.at[b]], sem_s[b]).wait()

        def zrow(r, carry):
            for v in range(dh // _LANES):
                rows[0, r, pl.ds(v * _LANES, _LANES)] = jnp.zeros(
                    (_LANES,), jnp.float32
                )
            return carry

        issue_meta(0, 0)
        issue_meta(1, 1)
        lax.fori_loop(0, _CHUNK, zrow, None)
        nz_full = rpt // _CHUNK
        for kz in range(nz_full):
            pltpu.sync_copy(rows.at[0], agg.at[pl.ds(rbase + kz * _CHUNK, _CHUNK)])
        zrem = rpt - nz_full * _CHUNK
        if zrem:
            pltpu.sync_copy(
                rows.at[0, pl.ds(0, zrem)],
                agg.at[pl.ds(rbase + nz_full * _CHUNK, zrem)],
            )
        if rem:
            @pl.when(s == _NS - 1)
            def _zero_tail():
                pltpu.sync_copy(rows.at[0, pl.ds(0, rem)], agg.at[pl.ds(tbase, rem)])
        plsc.subcore_barrier()

        wait_meta(0, 0)
        prep_gather(0, 0, 0)

        def fbody(t, carry):
            j0 = t * _SUPER
            for ks in range(_SUPER):
                j = j0 + ks
                b = ks % _NB
                bn = (ks + 1) % _NB
                m = ks // _G
                kk = ks % _G
                mn = ((ks + 1) // _G) % _NGB
                kkn = (ks + 1) % _G
                if ks <= 1:
                    @pl.when(j >= 2)
                    def _(bn=bn):
                        wait_scatter(bn)
                else:
                    wait_scatter(bn)
                if kkn == 0:
                    wait_meta(t * _NGB + ks // _G + 1, mn)
                prep_gather(bn, mn, kkn)
                if kk == 0:
                    gpre = t * _NGB + m + 2
                    gsl_pre = (m + 2) % _NGB
                    if m == _NGB - 1:
                        @pl.when(gpre < ng)
                        def _(gpre=gpre, gsl_pre=gsl_pre):
                            issue_meta(gpre, gsl_pre)
                    else:
                        issue_meta(gpre, gsl_pre)
                wait_gather(b)
                scale(b, m, kk)
                issue_scatter(b)
            return carry

        lax.fori_loop(0, nsteady // _SUPER, fbody, None)
        for j in range(nsteady, nch):
            b = j % _NB
            bn = (j + 1) % _NB
            kk = j % _G
            wait_scatter(bn)
            if j + 1 < nch:
                prep_gather(bn, (ng - 1) % _NGB, kk + 1)
            wait_gather(b)
            scale(b, (ng - 1) % _NGB, kk)
            issue_scatter(b)
        wait_scatter((nch - 2) % _NB)
        wait_scatter((nch - 1) % _NB)
        plsc.subcore_barrier()

        pltpu.sync_copy(agg.at[pl.ds(rbase, rpt)], out_hbm.at[c, pl.ds(rbase, rpt)])
        if rem:
            @pl.when(s == _NS - 1)
            def _write_tail():
                pltpu.sync_copy(agg.at[pl.ds(tbase, rem)], out_hbm.at[c, pl.ds(tbase, rem)])

    return k(xr, meta, wr)


def _dense_relu(agg, W):
    _, n, dh = agg.shape
    d_out = W.shape[1]
    bm = 1000

    def body(a_ref, w_ref, o_ref):
        a = a_ref[...]
        w = w_ref[...]
        y = jnp.dot(a[0], w[:dh], preferred_element_type=jnp.float32)
        y = y + jnp.dot(a[1], w[dh:], preferred_element_type=jnp.float32)
        o_ref[...] = jnp.maximum(y, 0.0)

    return pl.pallas_call(
        body,
        grid=(n // bm,),
        in_specs=[
            pl.BlockSpec((2, bm, dh), lambda i: (0, i, 0)),
            pl.BlockSpec(W.shape, lambda i: (0, 0)),
        ],
        out_specs=pl.BlockSpec((bm, d_out), lambda i: (i, 0)),
        out_shape=jax.ShapeDtypeStruct((n, d_out), jnp.float32),
    )(agg, W)


def kernel(x, edge_index, edge_weight, W):
    n, d = x.shape
    e = edge_weight.shape[0]
    dh = d // 2
    xr = x.reshape(2 * n, dh)
    ept = e // _NS
    gsz = _G * _CHUNK
    ng = -(-ept // gsz)
    eptp = ng * gsz
    ei = edge_index.astype(jnp.int32).reshape(2, _NS, ept)
    ei = jnp.concatenate(
        [ei, jnp.zeros((2, _NS, eptp - ept), jnp.int32)], axis=2
    )
    meta = ei.reshape(2 * _NS, ng, _G, _CHUNK)
    wv = edge_weight.reshape(_NS, ept)
    wv = jnp.concatenate(
        [wv, jnp.zeros((_NS, eptp - ept), jnp.float32)], axis=1
    )
    wr = wv.reshape(_NS, ng, _G, _CHUNK)
    agg = _spmm(xr, meta, wr, n)
    return _dense_relu(agg, W)
